# Initial kernel scaffold; baseline (speedup 1.0000x reference)
import jax
import jax.numpy as jnp
from jax.experimental import pallas as pl


def kernel(x, h, c, edge_index, W_iouf, U_iou, b_iou, U_f_W, U_f_b):
    raise NotImplementedError("write your pallas kernel here")


# speedup vs baseline: 4.0712x; 4.0712x over previous
import functools

import jax
import jax.numpy as jnp
from jax import lax
from jax.experimental import pallas as pl
from jax.experimental.pallas import tpu as pltpu
from jax.experimental.pallas import tpu_sc as plsc

N = 10000
E = 320000
DIM = 128


_ROWS = 400
_GRID = N // _ROWS


def _pre_body(x_ref, h_ref, c_ref, w3t_ref, uft_ref, ufb_ref, iou_ref, gc_ref):
    fh = jnp.dot(h_ref[...], uft_ref[...], preferred_element_type=jnp.float32)
    g = jax.nn.sigmoid(fh + ufb_ref[...])
    gc_ref[...] = g * c_ref[...]
    iou_ref[...] = jnp.dot(x_ref[...], w3t_ref[...],
                           preferred_element_type=jnp.float32)


def _pre(x, h, c, w3t, uft, ufb):
    return pl.pallas_call(
        _pre_body,
        grid=(_GRID,),
        in_specs=[
            pl.BlockSpec((_ROWS, DIM), lambda i: (i, 0)),
            pl.BlockSpec((_ROWS, DIM), lambda i: (i, 0)),
            pl.BlockSpec((_ROWS, DIM), lambda i: (i, 0)),
            pl.BlockSpec((DIM, 3 * DIM), lambda i: (0, 0)),
            pl.BlockSpec((DIM, DIM), lambda i: (0, 0)),
            pl.BlockSpec((1, DIM), lambda i: (0, 0)),
        ],
        out_specs=[
            pl.BlockSpec((_ROWS, 3 * DIM), lambda i: (i, 0)),
            pl.BlockSpec((_ROWS, DIM), lambda i: (i, 0)),
        ],
        out_shape=[
            jax.ShapeDtypeStruct((N, 3 * DIM), jnp.float32),
            jax.ShapeDtypeStruct((N, DIM), jnp.float32),
        ],
    )(x, h, c, w3t, uft, ufb)


_NSUB = 16
_EDGES_PER_SUB = E // _NSUB
_CHUNK = 80
_NCHUNK = _EDGES_PER_SUB // _CHUNK
_APAD = 10240
_ROWS_PER_SUB = _APAD // _NSUB
_ZROWS = 128

@functools.cache
def _build_sc_segment_sum():
    mesh = plsc.VectorSubcoreMesh(core_axis_name="c", subcore_axis_name="s")
    return pl.kernel(
        _sc_segment_sum_body,
        out_type=jax.ShapeDtypeStruct((2 * _APAD, DIM), jnp.float32),
        mesh=mesh,
        scratch_types=[
            pltpu.VMEM((_CHUNK,), jnp.int32),
            pltpu.VMEM((_CHUNK,), jnp.int32),
            pltpu.VMEM((_CHUNK, DIM), jnp.float32),
            pltpu.VMEM((_ZROWS, DIM), jnp.float32),
            pltpu.VMEM_SHARED((_APAD, DIM), jnp.float32),
            pltpu.SemaphoreType.DMA,
        ],
    )


def _sc_segment_sum_body(table_hbm, srcx_hbm, dst_hbm, out_hbm,
                         src_v, dst_v, rows_v, buf_v, acc, sem):
    c = lax.axis_index("c")
    s = lax.axis_index("s")

    def _zero_body(i, _):
        r = i // (DIM // 16)
        k = (i % (DIM // 16)) * 16
        buf_v[r, pl.ds(k, 16)] = jnp.zeros((16,), jnp.float32)
        return _

    lax.fori_loop(0, _ZROWS * (DIM // 16), _zero_body, None)

    def _zcopy_body(k, _):
        pltpu.sync_copy(buf_v,
                        acc.at[pl.ds(s * _ROWS_PER_SUB + k * _ZROWS, _ZROWS)])
        return _

    lax.fori_loop(0, _ROWS_PER_SUB // _ZROWS, _zcopy_body, None)
    plsc.subcore_barrier()

    def _edge_body(t, _):
        base = s * _EDGES_PER_SUB + t * _CHUNK
        pltpu.sync_copy(srcx_hbm.at[pl.ds(c * E + base, _CHUNK)], src_v)
        pltpu.sync_copy(dst_hbm.at[pl.ds(base, _CHUNK)], dst_v)
        pltpu.async_copy(table_hbm.at[src_v], rows_v, sem).wait()
        pltpu.sync_copy(rows_v, acc.at[dst_v], add=True)
        return _

    lax.fori_loop(0, _NCHUNK, _edge_body, None)
    plsc.subcore_barrier()

    def _wb_body(k, _):
        row = s * _ROWS_PER_SUB + k * _ZROWS
        pltpu.sync_copy(acc.at[pl.ds(row, _ZROWS)], buf_v)
        pltpu.sync_copy(buf_v, out_hbm.at[pl.ds(c * _APAD + row, _ZROWS)])
        return _

    lax.fori_loop(0, _ROWS_PER_SUB // _ZROWS, _wb_body, None)


def _post_body(hs_ref, ca_ref, ix_ref, ut_ref, b_ref, h_ref, c_ref):
    sm = jnp.dot(hs_ref[...], ut_ref[...], preferred_element_type=jnp.float32)
    iou = ix_ref[...] + sm + b_ref[...]
    i = jax.nn.sigmoid(iou[:, :DIM])
    o = jax.nn.sigmoid(iou[:, DIM:2 * DIM])
    u = jnp.tanh(iou[:, 2 * DIM:])
    c_new = i * u + ca_ref[...]
    h_ref[...] = o * jnp.tanh(c_new)
    c_ref[...] = c_new


def _post(h_sum, c_agg, iou_x, ut, b):
    return pl.pallas_call(
        _post_body,
        grid=(_GRID,),
        in_specs=[
            pl.BlockSpec((_ROWS, DIM), lambda i: (i, 0)),
            pl.BlockSpec((_ROWS, DIM), lambda i: (i, 0)),
            pl.BlockSpec((_ROWS, 3 * DIM), lambda i: (i, 0)),
            pl.BlockSpec((DIM, 3 * DIM), lambda i: (0, 0)),
            pl.BlockSpec((1, 3 * DIM), lambda i: (0, 0)),
        ],
        out_specs=[
            pl.BlockSpec((_ROWS, DIM), lambda i: (i, 0)),
            pl.BlockSpec((_ROWS, DIM), lambda i: (i, 0)),
        ],
        out_shape=[
            jax.ShapeDtypeStruct((N, DIM), jnp.float32),
            jax.ShapeDtypeStruct((N, DIM), jnp.float32),
        ],
    )(h_sum, c_agg, iou_x, ut, b)


@jax.jit
def kernel(x, h, c, edge_index, W_iouf, U_iou, b_iou, U_f_W, U_f_b):
    src = edge_index[0].astype(jnp.int32)
    dst = edge_index[1].astype(jnp.int32)

    w3t = W_iouf[:3 * DIM].T
    uft = U_f_W.T
    ufb = U_f_b.reshape(1, DIM)
    ut = U_iou.T

    iou_x, gc = _pre(x, h, c, w3t, uft, ufb)

    table = jnp.concatenate([h, gc], axis=0)
    srcx = jnp.concatenate([src, src + N], axis=0)

    agg = _build_sc_segment_sum()(table, srcx, dst)
    h_sum = agg[:N]
    c_agg = agg[_APAD:_APAD + N]

    h_new, c_new = _post(h_sum, c_agg, iou_x, ut, b_iou)
    return h_new, c_new


# speedup vs baseline: 7.2732x; 1.7865x over previous
import functools

import jax
import jax.numpy as jnp
from jax import lax
from jax.experimental import pallas as pl
from jax.experimental.pallas import tpu as pltpu
from jax.experimental.pallas import tpu_sc as plsc

N = 10000
E = 320000
DIM = 128


_ROWS = 400
_GRID = N // _ROWS


def _pre_body(x_ref, h_ref, c_ref, w3t_ref, uft_ref, ufb_ref, iou_ref, gc_ref):
    fh = jnp.dot(h_ref[...], uft_ref[...], preferred_element_type=jnp.float32)
    g = jax.nn.sigmoid(fh + ufb_ref[...])
    gc_ref[...] = g * c_ref[...]
    iou_ref[...] = jnp.dot(x_ref[...], w3t_ref[...],
                           preferred_element_type=jnp.float32)


def _pre(x, h, c, w3t, uft, ufb):
    return pl.pallas_call(
        _pre_body,
        grid=(_GRID,),
        in_specs=[
            pl.BlockSpec((_ROWS, DIM), lambda i: (i, 0)),
            pl.BlockSpec((_ROWS, DIM), lambda i: (i, 0)),
            pl.BlockSpec((_ROWS, DIM), lambda i: (i, 0)),
            pl.BlockSpec((DIM, 3 * DIM), lambda i: (0, 0)),
            pl.BlockSpec((DIM, DIM), lambda i: (0, 0)),
            pl.BlockSpec((1, DIM), lambda i: (0, 0)),
        ],
        out_specs=[
            pl.BlockSpec((_ROWS, 3 * DIM), lambda i: (i, 0)),
            pl.BlockSpec((_ROWS, DIM), lambda i: (i, 0)),
        ],
        out_shape=[
            jax.ShapeDtypeStruct((N, 3 * DIM), jnp.float32),
            jax.ShapeDtypeStruct((N, DIM), jnp.float32),
        ],
    )(x, h, c, w3t, uft, ufb)


_NSUB = 16
_CHUNK = 80
_NCHUNK = E // _NSUB // _CHUNK
_EDGES_PER_SUB = _NCHUNK * _CHUNK
_APAD = 10240
_ROWS_PER_SUB = _APAD // _NSUB

@functools.cache
def _build_sc_segment_sum():
    mesh = plsc.VectorSubcoreMesh(core_axis_name="c", subcore_axis_name="s")
    return pl.kernel(
        _sc_segment_sum_body,
        out_type=jax.ShapeDtypeStruct((2 * _APAD, DIM), jnp.float32),
        mesh=mesh,
        scratch_types=[
            pltpu.VMEM((_CHUNK,), jnp.int32),
            pltpu.VMEM((_CHUNK,), jnp.int32),
            pltpu.VMEM((_CHUNK,), jnp.int32),
            pltpu.VMEM((_CHUNK,), jnp.int32),
            pltpu.VMEM((_CHUNK, DIM), jnp.float32),
            pltpu.VMEM((_CHUNK, DIM), jnp.float32),
            pltpu.VMEM_SHARED((_APAD, DIM), jnp.float32),
        ] + [pltpu.SemaphoreType.DMA] * 8,
    )


def _sc_segment_sum_body(table_hbm, srcx_hbm, dst_hbm, out_hbm,
                         sbuf0, sbuf1, dbuf0, dbuf1, rows0, rows1, acc,
                         gsem0, gsem1, ssem0, ssem1, is0, is1, id0, id1):
    c = lax.axis_index("c")
    s = lax.axis_index("s")
    sbuf = (sbuf0, sbuf1)
    dbuf = (dbuf0, dbuf1)
    rows = (rows0, rows1)
    gsem = (gsem0, gsem1)
    ssem = (ssem0, ssem1)
    isem = (is0, is1)
    idsem = (id0, id1)
    sbase = c * E + s * _EDGES_PER_SUB
    dbase = s * _EDGES_PER_SUB

    def src_start(t, b):
        pltpu.async_copy(srcx_hbm.at[pl.ds(sbase + t * _CHUNK, _CHUNK)],
                         sbuf[b], isem[b])

    def src_wait(b):
        pltpu.make_async_copy(srcx_hbm.at[pl.ds(sbase, _CHUNK)],
                              sbuf[b], isem[b]).wait()

    def dst_start(t, b):
        pltpu.async_copy(dst_hbm.at[pl.ds(dbase + t * _CHUNK, _CHUNK)],
                         dbuf[b], idsem[b])

    def dst_wait(b):
        pltpu.make_async_copy(dst_hbm.at[pl.ds(dbase, _CHUNK)],
                              dbuf[b], idsem[b]).wait()

    def gather_start(b):
        pltpu.async_copy(table_hbm.at[sbuf[b]], rows[b], gsem[b])

    def gather_wait(b):
        pltpu.make_async_copy(table_hbm.at[sbuf[b]], rows[b], gsem[b]).wait()

    def _zero_body(i, _):
        r = i // (DIM // 16)
        k = (i % (DIM // 16)) * 16
        rows0[r, pl.ds(k, 16)] = jnp.zeros((16,), jnp.float32)
        return _

    lax.fori_loop(0, _CHUNK * (DIM // 16), _zero_body, None)

    def _zcopy_body(k, _):
        pltpu.sync_copy(rows0,
                        acc.at[pl.ds(s * _ROWS_PER_SUB + k * _CHUNK, _CHUNK)])
        return _

    lax.fori_loop(0, _ROWS_PER_SUB // _CHUNK, _zcopy_body, None)
    plsc.subcore_barrier()

    src_start(0, 0)
    dst_start(0, 0)
    src_start(1, 1)
    dst_start(1, 1)
    src_wait(0)
    dst_wait(0)
    gather_start(0)

    def _step(t, b, *, idx_next, gather_next):
        gather_wait(b)
        sc = pltpu.async_copy(rows[b], acc.at[dbuf[b]], ssem[b], add=True)
        if idx_next:
            src_start(t + 2, b)
        if gather_next:
            src_wait(1 - b)
            dst_wait(1 - b)
            gather_start(1 - b)
        sc.wait()
        if idx_next:
            dst_start(t + 2, b)

    def _edge_body(i, _):
        _step(i * 2, 0, idx_next=True, gather_next=True)
        _step(i * 2 + 1, 1, idx_next=True, gather_next=True)
        return _

    lax.fori_loop(0, _NCHUNK // 2 - 1, _edge_body, None)
    _step(_NCHUNK - 2, 0, idx_next=False, gather_next=True)
    _step(_NCHUNK - 1, 1, idx_next=False, gather_next=False)
    plsc.subcore_barrier()

    def _wb_body(k, _):
        row = s * _ROWS_PER_SUB + k * _CHUNK
        pltpu.sync_copy(acc.at[pl.ds(row, _CHUNK)], rows0)
        pltpu.sync_copy(rows0, out_hbm.at[pl.ds(c * _APAD + row, _CHUNK)])
        return _

    lax.fori_loop(0, _ROWS_PER_SUB // _CHUNK, _wb_body, None)


def _post_body(hs_ref, ca_ref, ix_ref, ut_ref, b_ref, h_ref, c_ref):
    sm = jnp.dot(hs_ref[...], ut_ref[...], preferred_element_type=jnp.float32)
    iou = ix_ref[...] + sm + b_ref[...]
    i = jax.nn.sigmoid(iou[:, :DIM])
    o = jax.nn.sigmoid(iou[:, DIM:2 * DIM])
    u = jnp.tanh(iou[:, 2 * DIM:])
    c_new = i * u + ca_ref[...]
    h_ref[...] = o * jnp.tanh(c_new)
    c_ref[...] = c_new


def _post(h_sum, c_agg, iou_x, ut, b):
    return pl.pallas_call(
        _post_body,
        grid=(_GRID,),
        in_specs=[
            pl.BlockSpec((_ROWS, DIM), lambda i: (i, 0)),
            pl.BlockSpec((_ROWS, DIM), lambda i: (i, 0)),
            pl.BlockSpec((_ROWS, 3 * DIM), lambda i: (i, 0)),
            pl.BlockSpec((DIM, 3 * DIM), lambda i: (0, 0)),
            pl.BlockSpec((1, 3 * DIM), lambda i: (0, 0)),
        ],
        out_specs=[
            pl.BlockSpec((_ROWS, DIM), lambda i: (i, 0)),
            pl.BlockSpec((_ROWS, DIM), lambda i: (i, 0)),
        ],
        out_shape=[
            jax.ShapeDtypeStruct((N, DIM), jnp.float32),
            jax.ShapeDtypeStruct((N, DIM), jnp.float32),
        ],
    )(h_sum, c_agg, iou_x, ut, b)


@jax.jit
def kernel(x, h, c, edge_index, W_iouf, U_iou, b_iou, U_f_W, U_f_b):
    src = edge_index[0].astype(jnp.int32)
    dst = edge_index[1].astype(jnp.int32)

    w3t = W_iouf[:3 * DIM].T
    uft = U_f_W.T
    ufb = U_f_b.reshape(1, DIM)
    ut = U_iou.T

    iou_x, gc = _pre(x, h, c, w3t, uft, ufb)

    table = jnp.concatenate([h, gc], axis=0)
    srcx = jnp.concatenate([src, src + N], axis=0)

    agg = _build_sc_segment_sum()(table, srcx, dst)
    h_sum = agg[:N]
    c_agg = agg[_APAD:_APAD + N]

    h_new, c_new = _post(h_sum, c_agg, iou_x, ut, b_iou)
    return h_new, c_new
